# Initial kernel scaffold; baseline (speedup 1.0000x reference)
import functools

import jax
import jax.numpy as jnp
from jax import lax
from jax.experimental import pallas as pl
from jax.experimental.pallas import tpu as pltpu
from jax.experimental.pallas import tpu_sc as plsc

N = 100000
E = 3200000
T = 4

NC = 2
NS = 16
C = 128
K = 8
NBLK = 98
PT = C * K * NBLK
EP = NC * NS * PT
NP = 100352
NP16 = NP // NS
RB = NP * T // 128


def _edge_pass_body(with_deg, *refs):
    if with_deg:
        (x_hbm, src_hbm, dst_hbm, z4_hbm, z1_hbm, ones_hbm,
         s_out, deg_out,
         acc_sp, deg_sp, src_blk, dst_blk, rows, ones_v,
         gsem, ssem, dsem) = refs
    else:
        (x_hbm, src_hbm, dst_hbm, z4_hbm,
         s_out,
         acc_sp, src_blk, dst_blk, rows,
         gsem, ssem) = refs

    c = lax.axis_index("c")
    s = lax.axis_index("s")
    row0 = s * NP16

    pltpu.sync_copy(z4_hbm.at[pl.ds(row0, NP16)], acc_sp.at[pl.ds(row0, NP16)])
    if with_deg:
        pltpu.sync_copy(z1_hbm.at[pl.ds(row0, NP16)], deg_sp.at[pl.ds(row0, NP16)])
        pltpu.sync_copy(ones_hbm, ones_v)
    plsc.subcore_barrier()

    def block(nb, carry):
        pltpu.sync_copy(src_hbm.at[c, s, nb], src_blk)
        pltpu.sync_copy(dst_hbm.at[c, s, nb], dst_blk)
        gds = []
        for j in range(K):
            gds.append(pltpu.async_copy(
                x_hbm.at[src_blk.at[j]], rows.at[j], gsem))
        for d in gds:
            d.wait()
        sds = []
        for j in range(K):
            sds.append(pltpu.async_copy(
                rows.at[j], acc_sp.at[dst_blk.at[j]], ssem, add=True))
        if with_deg:
            for j in range(K):
                sds.append(pltpu.async_copy(
                    ones_v, deg_sp.at[src_blk.at[j]], dsem, add=True))
        for d in sds:
            d.wait()
        return carry

    lax.fori_loop(0, NBLK, block, 0)
    plsc.subcore_barrier()

    pltpu.sync_copy(acc_sp.at[pl.ds(row0, NP16)], s_out.at[c, pl.ds(row0, NP16)])
    if with_deg:
        pltpu.sync_copy(deg_sp.at[pl.ds(row0, NP16)],
                        deg_out.at[c, pl.ds(row0, NP16)])


def _make_edge_pass(with_deg):
    mesh = plsc.VectorSubcoreMesh(core_axis_name="c", subcore_axis_name="s")
    if with_deg:
        out_type = (jax.ShapeDtypeStruct((NC, NP, T), jnp.float32),
                    jax.ShapeDtypeStruct((NC, NP, 1), jnp.float32))
        scratch = [
            pltpu.VMEM_SHARED((NP, T), jnp.float32),
            pltpu.VMEM_SHARED((NP, 1), jnp.float32),
            pltpu.VMEM((K, C), jnp.int32),
            pltpu.VMEM((K, C), jnp.int32),
            pltpu.VMEM((K, C, T), jnp.float32),
            pltpu.VMEM((C, 1), jnp.float32),
            pltpu.SemaphoreType.DMA,
            pltpu.SemaphoreType.DMA,
            pltpu.SemaphoreType.DMA,
        ]
    else:
        out_type = jax.ShapeDtypeStruct((NC, NP, T), jnp.float32)
        scratch = [
            pltpu.VMEM_SHARED((NP, T), jnp.float32),
            pltpu.VMEM((K, C), jnp.int32),
            pltpu.VMEM((K, C), jnp.int32),
            pltpu.VMEM((K, C, T), jnp.float32),
            pltpu.SemaphoreType.DMA,
            pltpu.SemaphoreType.DMA,
        ]
    return pl.kernel(functools.partial(_edge_pass_body, with_deg),
                     out_type=out_type, mesh=mesh, scratch_types=scratch)


def _combine_body(a1_ref, g_ref, b_ref, x_ref, sa_ref, sb_ref, da_ref, db_ref,
                  o_ref):
    a1 = a1_ref[0, 0]
    g = g_ref[0, 0]
    b = b_ref[0, 0]
    p = jax.nn.sigmoid(g)
    sw = jnp.exp(a1)
    nw = sw * jnp.tanh(a1)
    deg = jnp.maximum(da_ref[...] + db_ref[...], 1.0)
    ld = jnp.log(deg)
    o_ref[...] = (sw * jnp.exp(p * ld) * x_ref[...]
                  + nw * jnp.exp((p - 1.0) * ld) * (sa_ref[...] + sb_ref[...])
                  + b)


_combine = pl.pallas_call(
    _combine_body,
    out_shape=jax.ShapeDtypeStruct((RB, 128), jnp.float32),
    in_specs=[pl.BlockSpec(memory_space=pltpu.SMEM)] * 3
             + [pl.BlockSpec((RB, 128), lambda: (0, 0))] * 5,
)


def _expand_deg(deg_part):
    return jnp.broadcast_to(deg_part, (NP, T)).reshape(RB, 128)


@jax.jit
def kernel(x, edge_index, alpha1_0, alpha2_0, gamma_0, bias_0,
           alpha1_1, alpha2_1, gamma_1, bias_1):
    src = edge_index[0]
    dst = edge_index[1]
    pad = EP - E
    srcp = jnp.concatenate(
        [src, jnp.full((pad,), N, jnp.int32)]).reshape(NC, NS, NBLK, K, C)
    dstp = jnp.concatenate(
        [dst, jnp.full((pad,), N, jnp.int32)]).reshape(NC, NS, NBLK, K, C)
    x2d = jnp.zeros((NP, T), jnp.float32).at[:N, :].set(x.T)
    z4 = jnp.zeros((NP, T), jnp.float32)
    z1 = jnp.zeros((NP, 1), jnp.float32)
    ones_c = jnp.ones((C, 1), jnp.float32)

    s0, deg = _make_edge_pass(True)(x2d, srcp, dstp, z4, z1, ones_c)
    da = _expand_deg(deg[0])
    db = _expand_deg(deg[1])

    xf = x2d.reshape(RB, 128)
    y1f = _combine(alpha1_0, gamma_0, bias_0, xf,
                   s0[0].reshape(RB, 128), s0[1].reshape(RB, 128), da, db)

    y1 = y1f.reshape(NP, T)
    s1 = _make_edge_pass(False)(y1, srcp, dstp, z4)
    y2f = _combine(alpha1_1, gamma_1, bias_1, y1f,
                   s1[0].reshape(RB, 128), s1[1].reshape(RB, 128), da, db)

    return y2f.reshape(NP, T)[:N, :].T


# speedup vs baseline: 43.9784x; 43.9784x over previous
import functools

import jax
import jax.numpy as jnp
from jax import lax
from jax.experimental import pallas as pl
from jax.experimental.pallas import tpu as pltpu
from jax.experimental.pallas import tpu_sc as plsc

N = 100000
E = 3200000
T = 4
D = 16

NC = 2
NS = 16
C = 128
K = 8
NBLK = 98
PT = C * K * NBLK
EP = NC * NS * PT
NP = 100352
NP16 = NP // NS
RW = NP * D // 128
GB = 8
RWB = RW // GB


def _edge_pass_body(with_deg, *refs):
    if with_deg:
        (x_hbm, src_hbm, dst_hbm, z_hbm,
         s_out,
         acc_sp, src_blk, dst_blk, rows, cones,
         gsem, ssem, dsem) = refs
    else:
        (x_hbm, src_hbm, dst_hbm, z_hbm,
         s_out,
         acc_sp, src_blk, dst_blk, rows,
         gsem, ssem) = refs

    c = lax.axis_index("c")
    s = lax.axis_index("s")
    row0 = s * NP16

    pltpu.sync_copy(z_hbm.at[pl.ds(row0, NP16)], acc_sp.at[pl.ds(row0, NP16)])
    if with_deg:
        cvec = jnp.where(lax.iota(jnp.int32, 16) == 4,
                         jnp.float32(1.0), jnp.float32(0.0))
        for i in range(C):
            cones[i, :] = cvec
    plsc.subcore_barrier()

    def block(nb, carry):
        pltpu.sync_copy(src_hbm.at[c, s, nb], src_blk)
        pltpu.sync_copy(dst_hbm.at[c, s, nb], dst_blk)
        gds = []
        for j in range(K):
            gds.append(pltpu.async_copy(
                x_hbm.at[src_blk.at[j]], rows.at[j], gsem))
        for d in gds:
            d.wait()
        sds = []
        for j in range(K):
            sds.append(pltpu.async_copy(
                rows.at[j], acc_sp.at[dst_blk.at[j]], ssem, add=True))
        if with_deg:
            for j in range(K):
                sds.append(pltpu.async_copy(
                    cones, acc_sp.at[src_blk.at[j]], dsem, add=True))
        for d in sds:
            d.wait()
        return carry

    lax.fori_loop(0, NBLK, block, 0)
    plsc.subcore_barrier()

    pltpu.sync_copy(acc_sp.at[pl.ds(row0, NP16)], s_out.at[c, pl.ds(row0, NP16)])


def _make_edge_pass(with_deg):
    mesh = plsc.VectorSubcoreMesh(core_axis_name="c", subcore_axis_name="s")
    scratch = [
        pltpu.VMEM_SHARED((NP, D), jnp.float32),
        pltpu.VMEM((K, C), jnp.int32),
        pltpu.VMEM((K, C), jnp.int32),
        pltpu.VMEM((K, C, D), jnp.float32),
    ]
    if with_deg:
        scratch += [pltpu.VMEM((C, D), jnp.float32),
                    pltpu.SemaphoreType.DMA,
                    pltpu.SemaphoreType.DMA,
                    pltpu.SemaphoreType.DMA]
    else:
        scratch += [pltpu.SemaphoreType.DMA,
                    pltpu.SemaphoreType.DMA]
    return pl.kernel(functools.partial(_edge_pass_body, with_deg),
                     out_type=jax.ShapeDtypeStruct((NC, NP, D), jnp.float32),
                     mesh=mesh, scratch_types=scratch,
                     compiler_params=pltpu.CompilerParams(
                         use_tc_tiling_on_sc=False))


def _combine_body(a1_ref, g_ref, b_ref, x_ref, sa_ref, sb_ref, da_ref, db_ref,
                  o_ref):
    a1 = a1_ref[0, 0]
    g = g_ref[0, 0]
    b = b_ref[0, 0]
    p = jax.nn.sigmoid(g)
    sw = jnp.exp(a1)
    nw = sw * jnp.tanh(a1)
    deg = jnp.maximum(da_ref[...] + db_ref[...], 1.0)
    ld = jnp.log(deg)
    o_ref[...] = (sw * jnp.exp(p * ld) * x_ref[...]
                  + nw * jnp.exp((p - 1.0) * ld) * (sa_ref[...] + sb_ref[...])
                  + b)


_combine = pl.pallas_call(
    _combine_body,
    grid=(GB,),
    out_shape=jax.ShapeDtypeStruct((RW, 128), jnp.float32),
    in_specs=[pl.BlockSpec(memory_space=pltpu.SMEM)] * 3
             + [pl.BlockSpec((RWB, 128), lambda i: (i, 0))] * 5,
    out_specs=pl.BlockSpec((RWB, 128), lambda i: (i, 0)),
)


def _expand_deg(s_part):
    return jnp.broadcast_to(s_part[:, 4:5], (NP, D)).reshape(RW, 128)


@jax.jit
def kernel(x, edge_index, alpha1_0, alpha2_0, gamma_0, bias_0,
           alpha1_1, alpha2_1, gamma_1, bias_1):
    src = edge_index[0]
    dst = edge_index[1]
    pad = EP - E
    srcp = jnp.concatenate(
        [src, jnp.full((pad,), N, jnp.int32)]).reshape(NC, NS, NBLK, K, C)
    dstp = jnp.concatenate(
        [dst, jnp.full((pad,), N, jnp.int32)]).reshape(NC, NS, NBLK, K, C)
    x2d = jnp.zeros((NP, D), jnp.float32).at[:N, :T].set(x.T)
    z = jnp.zeros((NP, D), jnp.float32)

    s0 = _make_edge_pass(True)(x2d, srcp, dstp, z)
    da = _expand_deg(s0[0])
    db = _expand_deg(s0[1])

    xf = x2d.reshape(RW, 128)
    y1f = _combine(alpha1_0, gamma_0, bias_0, xf,
                   s0[0].reshape(RW, 128), s0[1].reshape(RW, 128), da, db)

    y1 = y1f.reshape(NP, D)
    s1 = _make_edge_pass(False)(y1, srcp, dstp, z)
    y2f = _combine(alpha1_1, gamma_1, bias_1, y1f,
                   s1[0].reshape(RW, 128), s1[1].reshape(RW, 128), da, db)

    return y2f.reshape(NP, D)[:N, :T].T


# speedup vs baseline: 57.2762x; 1.3024x over previous
import functools

import jax
import jax.numpy as jnp
from jax import lax
from jax.experimental import pallas as pl
from jax.experimental.pallas import tpu as pltpu
from jax.experimental.pallas import tpu_sc as plsc

N = 100000
E = 3200000
T = 4
D = 16

NC = 2
NS = 16
NW = NC * NS
C = 128
K = 4
NCHUNK = E // C
CPT = NCHUNK // NW
XTRA = NCHUNK - CPT * NW
NBF = CPT // K
CREM = CPT - NBF * K
NP = 100352
NP16 = NP // NS
RW = NP * D // 128
GB = 8
RWB = RW // GB

IBYTES = 2 * K * C * 4
GBYTES = K * C * D * 4


def _edge_pass_body(with_deg, *refs):
    if with_deg:
        (x_hbm, src_hbm, dst_hbm, z_hbm, zdr_hbm,
         s_out,
         acc_sp, src_blk, dst_blk, rows, tail_s, tail_d, cones,
         isem, gsem, ssem) = refs
    else:
        (x_hbm, src_hbm, dst_hbm, z_hbm, zdr_hbm,
         s_out,
         acc_sp, src_blk, dst_blk, rows, tail_s, tail_d,
         isem, gsem, ssem) = refs
        cones = None

    def drain_scatters(_):
        pltpu.make_async_copy(zdr_hbm.at[0], rows.at[0], ssem).wait()
        if with_deg:
            pltpu.make_async_copy(zdr_hbm.at[0], rows.at[0], ssem).wait()

    c = lax.axis_index("c")
    s = lax.axis_index("s")
    wid = c * NS + s
    row0 = s * NP16
    cb0 = wid * CPT

    pltpu.sync_copy(z_hbm.at[pl.ds(row0, NP16)], acc_sp.at[pl.ds(row0, NP16)])
    if with_deg:
        cvec = jnp.where(lax.iota(jnp.int32, 16) == 4,
                         jnp.float32(1.0), jnp.float32(0.0))
        for i in range(C):
            cones[i, :] = cvec
    plsc.subcore_barrier()

    pltpu.async_copy(src_hbm.at[pl.ds(cb0, K)], src_blk.at[0], isem)
    pltpu.async_copy(dst_hbm.at[pl.ds(cb0, K)], dst_blk.at[0], isem)

    def block(nb, carry):
        b2 = lax.rem(nb, 2)
        b3 = lax.rem(nb, 3)
        pltpu.make_async_copy(src_hbm.at[pl.ds(cb0, K)],
                              src_blk.at[b3], isem).wait()
        pltpu.make_async_copy(src_hbm.at[pl.ds(cb0, K)],
                              dst_blk.at[b3], isem).wait()

        pl.when(nb >= 2)(lambda: drain_scatters(0))

        for j in range(K):
            pltpu.async_copy(x_hbm.at[src_blk.at[b3, j]],
                             rows.at[b2, j], gsem)

        @pl.when(nb + 1 < NBF)
        def _():
            nxt = cb0 + (nb + 1) * K
            b3n = lax.rem(nb + 1, 3)
            pltpu.async_copy(src_hbm.at[pl.ds(nxt, K)], src_blk.at[b3n], isem)
            pltpu.async_copy(dst_hbm.at[pl.ds(nxt, K)], dst_blk.at[b3n], isem)

        pltpu.make_async_copy(zdr_hbm.at[0], rows.at[0], gsem).wait()
        for j in range(K):
            pltpu.async_copy(rows.at[b2, j],
                             acc_sp.at[dst_blk.at[b3, j]], ssem, add=True)
        if with_deg:
            for j in range(K):
                pltpu.async_copy(cones,
                                 acc_sp.at[src_blk.at[b3, j]], ssem, add=True)
        return carry

    lax.fori_loop(0, NBF, block, 0)
    drain_scatters(0)
    drain_scatters(0)

    for j in range(K):
        for i in range(C // 16):
            dummy = N + jnp.bitwise_and(i * 16 + lax.iota(jnp.int32, 16), 255)
            tail_s[j, pl.ds(i * 16, 16)] = dummy
            tail_d[j, pl.ds(i * 16, 16)] = dummy
    pltpu.sync_copy(src_hbm.at[pl.ds(cb0 + NBF * K, CREM)],
                    tail_s.at[pl.ds(0, CREM)])
    pltpu.sync_copy(dst_hbm.at[pl.ds(cb0 + NBF * K, CREM)],
                    tail_d.at[pl.ds(0, CREM)])

    @pl.when(wid < XTRA)
    def _():
        xc = NW * CPT + wid
        pltpu.sync_copy(src_hbm.at[pl.ds(xc, 1)], tail_s.at[pl.ds(CREM, 1)])
        pltpu.sync_copy(dst_hbm.at[pl.ds(xc, 1)], tail_d.at[pl.ds(CREM, 1)])

    gds = []
    for j in range(K):
        gds.append(pltpu.async_copy(x_hbm.at[tail_s.at[j]],
                                    rows.at[0, j], gsem))
    for d in gds:
        d.wait()
    sds = []
    for j in range(K):
        sds.append(pltpu.async_copy(rows.at[0, j],
                                    acc_sp.at[tail_d.at[j]], ssem, add=True))
    if with_deg:
        for j in range(K):
            sds.append(pltpu.async_copy(cones,
                                        acc_sp.at[tail_s.at[j]], ssem,
                                        add=True))
    for d in sds:
        d.wait()

    plsc.subcore_barrier()

    pltpu.sync_copy(acc_sp.at[pl.ds(row0, NP16)], s_out.at[c, pl.ds(row0, NP16)])


def _make_edge_pass(with_deg):
    mesh = plsc.VectorSubcoreMesh(core_axis_name="c", subcore_axis_name="s")
    scratch = [
        pltpu.VMEM_SHARED((NP, D), jnp.float32),
        pltpu.VMEM((3, K, C), jnp.int32),
        pltpu.VMEM((3, K, C), jnp.int32),
        pltpu.VMEM((2, K, C, D), jnp.float32),
        pltpu.VMEM((K, C), jnp.int32),
        pltpu.VMEM((K, C), jnp.int32),
    ]
    if with_deg:
        scratch.append(pltpu.VMEM((C, D), jnp.float32))
    scratch += [pltpu.SemaphoreType.DMA,
                pltpu.SemaphoreType.DMA,
                pltpu.SemaphoreType.DMA]
    return pl.kernel(functools.partial(_edge_pass_body, with_deg),
                     out_type=jax.ShapeDtypeStruct((NC, NP, D), jnp.float32),
                     mesh=mesh, scratch_types=scratch,
                     compiler_params=pltpu.CompilerParams(
                         use_tc_tiling_on_sc=False))


def _combine_body(a1_ref, g_ref, b_ref, x_ref, sa_ref, sb_ref, da_ref, db_ref,
                  o_ref):
    a1 = a1_ref[0, 0]
    g = g_ref[0, 0]
    b = b_ref[0, 0]
    p = jax.nn.sigmoid(g)
    sw = jnp.exp(a1)
    nw = sw * jnp.tanh(a1)
    deg = jnp.maximum(da_ref[...] + db_ref[...], 1.0)
    ld = jnp.log(deg)
    o_ref[...] = (sw * jnp.exp(p * ld) * x_ref[...]
                  + nw * jnp.exp((p - 1.0) * ld) * (sa_ref[...] + sb_ref[...])
                  + b)


_combine = pl.pallas_call(
    _combine_body,
    grid=(GB,),
    out_shape=jax.ShapeDtypeStruct((RW, 128), jnp.float32),
    in_specs=[pl.BlockSpec(memory_space=pltpu.SMEM)] * 3
             + [pl.BlockSpec((RWB, 128), lambda i: (i, 0))] * 5,
    out_specs=pl.BlockSpec((RWB, 128), lambda i: (i, 0)),
)


def _expand_deg(s_part):
    return jnp.broadcast_to(s_part[:, 4:5], (NP, D)).reshape(RW, 128)


@jax.jit
def kernel(x, edge_index, alpha1_0, alpha2_0, gamma_0, bias_0,
           alpha1_1, alpha2_1, gamma_1, bias_1):
    src2 = edge_index[0].reshape(NCHUNK, C)
    dst2 = edge_index[1].reshape(NCHUNK, C)
    x2d = jnp.zeros((NP, D), jnp.float32).at[:N, :T].set(x.T)
    z = jnp.zeros((NP, D), jnp.float32)
    zdr = jnp.zeros((1, K, C, D), jnp.float32)

    s0 = _make_edge_pass(True)(x2d, src2, dst2, z, zdr)
    da = _expand_deg(s0[0])
    db = _expand_deg(s0[1])

    xf = x2d.reshape(RW, 128)
    y1f = _combine(alpha1_0, gamma_0, bias_0, xf,
                   s0[0].reshape(RW, 128), s0[1].reshape(RW, 128), da, db)

    y1 = y1f.reshape(NP, D)
    s1 = _make_edge_pass(False)(y1, src2, dst2, z, zdr)
    y2f = _combine(alpha1_1, gamma_1, bias_1, y1f,
                   s1[0].reshape(RW, 128), s1[1].reshape(RW, 128), da, db)

    return y2f.reshape(NP, D)[:N, :T].T


# speedup vs baseline: 97.8341x; 1.7081x over previous
import functools

import jax
import jax.numpy as jnp
from jax import lax
from jax.experimental import pallas as pl
from jax.experimental.pallas import tpu as pltpu
from jax.experimental.pallas import tpu_sc as plsc

N = 100000
E = 3200000
T = 4
D = 16

NC = 2
NS = 16
NW = NC * NS
C = 128
K = 4
NCHUNK = E // C
CPT = NCHUNK // NW
XTRA = NCHUNK - CPT * NW
NBF = CPT // K
CREM = CPT - NBF * K
NP = 100352
NP16 = NP // NS
RW = NP * D // 128

IBYTES = 2 * K * C * 4
GBYTES = K * C * D * 4

NTR = NP // NW
RCH = 448
NCH = NTR // RCH
LN2 = 0.6931471805599453


def _edge_pass_body(with_deg, *refs):
    if with_deg:
        (x_hbm, src_hbm, dst_hbm, z_hbm, zdr_hbm,
         s_out,
         acc_sp, src_blk, dst_blk, rows, tail_s, tail_d, cones,
         isem, gsem, ssem) = refs
    else:
        (x_hbm, src_hbm, dst_hbm, z_hbm, zdr_hbm,
         s_out,
         acc_sp, src_blk, dst_blk, rows, tail_s, tail_d,
         isem, gsem, ssem) = refs
        cones = None

    def drain_scatters(_):
        pltpu.make_async_copy(zdr_hbm.at[0], rows.at[0], ssem).wait()
        if with_deg:
            pltpu.make_async_copy(zdr_hbm.at[0], rows.at[0], ssem).wait()

    c = lax.axis_index("c")
    s = lax.axis_index("s")
    wid = c * NS + s
    row0 = s * NP16
    cb0 = wid * CPT

    pltpu.sync_copy(z_hbm.at[pl.ds(row0, NP16)], acc_sp.at[pl.ds(row0, NP16)])
    if with_deg:
        cvec = jnp.where(lax.iota(jnp.int32, 16) == 4,
                         jnp.float32(1.0), jnp.float32(0.0))
        for i in range(C):
            cones[i, :] = cvec
    plsc.subcore_barrier()

    pltpu.async_copy(src_hbm.at[pl.ds(cb0, K)], src_blk.at[0], isem)
    pltpu.async_copy(dst_hbm.at[pl.ds(cb0, K)], dst_blk.at[0], isem)

    def block(nb, carry):
        b2 = lax.rem(nb, 2)
        b3 = lax.rem(nb, 3)
        pltpu.make_async_copy(src_hbm.at[pl.ds(cb0, K)],
                              src_blk.at[b3], isem).wait()
        pltpu.make_async_copy(src_hbm.at[pl.ds(cb0, K)],
                              dst_blk.at[b3], isem).wait()

        pl.when(nb >= 2)(lambda: drain_scatters(0))

        for j in range(K):
            pltpu.async_copy(x_hbm.at[src_blk.at[b3, j]],
                             rows.at[b2, j], gsem)

        @pl.when(nb + 1 < NBF)
        def _():
            nxt = cb0 + (nb + 1) * K
            b3n = lax.rem(nb + 1, 3)
            pltpu.async_copy(src_hbm.at[pl.ds(nxt, K)], src_blk.at[b3n], isem)
            pltpu.async_copy(dst_hbm.at[pl.ds(nxt, K)], dst_blk.at[b3n], isem)

        pltpu.make_async_copy(zdr_hbm.at[0], rows.at[0], gsem).wait()
        for j in range(K):
            pltpu.async_copy(rows.at[b2, j],
                             acc_sp.at[dst_blk.at[b3, j]], ssem, add=True)
        if with_deg:
            for j in range(K):
                pltpu.async_copy(cones,
                                 acc_sp.at[src_blk.at[b3, j]], ssem, add=True)
        return carry

    lax.fori_loop(0, NBF, block, 0)
    drain_scatters(0)
    drain_scatters(0)

    for j in range(K):
        for i in range(C // 16):
            dummy = N + jnp.bitwise_and(i * 16 + lax.iota(jnp.int32, 16), 255)
            tail_s[j, pl.ds(i * 16, 16)] = dummy
            tail_d[j, pl.ds(i * 16, 16)] = dummy
    pltpu.sync_copy(src_hbm.at[pl.ds(cb0 + NBF * K, CREM)],
                    tail_s.at[pl.ds(0, CREM)])
    pltpu.sync_copy(dst_hbm.at[pl.ds(cb0 + NBF * K, CREM)],
                    tail_d.at[pl.ds(0, CREM)])

    @pl.when(wid < XTRA)
    def _():
        xc = NW * CPT + wid
        pltpu.sync_copy(src_hbm.at[pl.ds(xc, 1)], tail_s.at[pl.ds(CREM, 1)])
        pltpu.sync_copy(dst_hbm.at[pl.ds(xc, 1)], tail_d.at[pl.ds(CREM, 1)])

    gds = []
    for j in range(K):
        gds.append(pltpu.async_copy(x_hbm.at[tail_s.at[j]],
                                    rows.at[0, j], gsem))
    for d in gds:
        d.wait()
    sds = []
    for j in range(K):
        sds.append(pltpu.async_copy(rows.at[0, j],
                                    acc_sp.at[tail_d.at[j]], ssem, add=True))
    if with_deg:
        for j in range(K):
            sds.append(pltpu.async_copy(cones,
                                        acc_sp.at[tail_s.at[j]], ssem,
                                        add=True))
    for d in sds:
        d.wait()

    plsc.subcore_barrier()

    pltpu.sync_copy(acc_sp.at[pl.ds(row0, NP16)], s_out.at[c, pl.ds(row0, NP16)])


def _make_edge_pass(with_deg):
    mesh = plsc.VectorSubcoreMesh(core_axis_name="c", subcore_axis_name="s")
    scratch = [
        pltpu.VMEM_SHARED((NP, D), jnp.float32),
        pltpu.VMEM((3, K, C), jnp.int32),
        pltpu.VMEM((3, K, C), jnp.int32),
        pltpu.VMEM((2, K, C, D), jnp.float32),
        pltpu.VMEM((K, C), jnp.int32),
        pltpu.VMEM((K, C), jnp.int32),
    ]
    if with_deg:
        scratch.append(pltpu.VMEM((C, D), jnp.float32))
    scratch += [pltpu.SemaphoreType.DMA,
                pltpu.SemaphoreType.DMA,
                pltpu.SemaphoreType.DMA]
    return pl.kernel(functools.partial(_edge_pass_body, with_deg),
                     out_type=jax.ShapeDtypeStruct((NC, NP, D), jnp.float32),
                     mesh=mesh, scratch_types=scratch,
                     compiler_params=pltpu.CompilerParams(
                         use_tc_tiling_on_sc=False))


def _ln(d):
    bits = plsc.bitcast(d, jnp.int32)
    e = (bits >> 23) - 127
    m = plsc.bitcast(jnp.bitwise_or(jnp.bitwise_and(bits, 0x007FFFFF),
                                    0x3F800000), jnp.float32)
    r = (m - 1.0) / (m + 1.0)
    r2 = r * r
    at = r * (1.0 + r2 * (1.0 / 3.0 + r2 * (1.0 / 5.0 + r2 * (1.0 / 7.0
                                                              + r2 / 9.0))))
    return e.astype(jnp.float32) * LN2 + 2.0 * at


def _combine_body(transpose_out, pv_hbm, x_hbm, sm_hbm, sd_hbm, out_hbm,
                  pbuf, xb, m0, m1, d0, d1, yb, wsb, wnb, sem):
    c = lax.axis_index("c")
    s = lax.axis_index("s")
    wid = c * NS + s
    base0 = wid * NTR

    pltpu.sync_copy(pv_hbm, pbuf)
    pvv = pbuf[...]
    a1v = jnp.full((16,), pvv[0], jnp.float32)
    gv = jnp.full((16,), pvv[1], jnp.float32)
    bv = jnp.full((16,), pvv[2], jnp.float32)
    pv = 1.0 / (1.0 + jnp.exp(-gv))
    swv = jnp.exp(a1v)
    e2a = jnp.exp(2.0 * a1v)
    nwv = swv * (e2a - 1.0) / (e2a + 1.0)

    idc4 = jnp.full((16,), 4, jnp.int32)
    iot = lax.iota(jnp.int32, 16)

    def chunk(ch, carry):
        base = base0 + ch * RCH
        cps = [pltpu.async_copy(x_hbm.at[pl.ds(base, RCH)], xb, sem),
               pltpu.async_copy(sm_hbm.at[0, pl.ds(base, RCH)], m0, sem),
               pltpu.async_copy(sm_hbm.at[1, pl.ds(base, RCH)], m1, sem),
               pltpu.async_copy(sd_hbm.at[0, pl.ds(base, RCH)], d0, sem),
               pltpu.async_copy(sd_hbm.at[1, pl.ds(base, RCH)], d1, sem)]
        for cp in cps:
            cp.wait()
        for gi in range(RCH // 16):
            idr = gi * 16 + iot
            dv = (plsc.load_gather(d0, [idr, idc4])
                  + plsc.load_gather(d1, [idr, idc4]))
            lnd = _ln(jnp.maximum(dv, 1.0))
            wsb[pl.ds(gi * 16, 16)] = swv * jnp.exp(pv * lnd)
            wnb[pl.ds(gi * 16, 16)] = nwv * jnp.exp((pv - 1.0) * lnd)

        def rowgrp(gi2, cr):
            r0 = gi2 * 16
            wsv = wsb[pl.ds(r0, 16)]
            wnv = wnb[pl.ds(r0, 16)]
            for l in range(16):
                i = r0 + l
                srow = m0[i, :] + m1[i, :]
                ws = jnp.full((16,), wsv[l], jnp.float32)
                wn = jnp.full((16,), wnv[l], jnp.float32)
                yb[i, :] = ws * xb[i, :] + wn * srow + bv
            return cr

        lax.fori_loop(0, RCH // 16, rowgrp, 0)

        if not transpose_out:
            pltpu.sync_copy(yb, out_hbm.at[pl.ds(base, RCH)])
        else:
            for t in range(T):
                idct = jnp.full((16,), t, jnp.int32)
                for gi in range(RCH // 16):
                    idr = gi * 16 + iot
                    wsb[pl.ds(gi * 16, 16)] = plsc.load_gather(yb, [idr, idct])
                pltpu.sync_copy(wsb, out_hbm.at[t, pl.ds(base, RCH)])
        return carry

    lax.fori_loop(0, NCH, chunk, 0)


def _make_combine(transpose_out):
    mesh = plsc.VectorSubcoreMesh(core_axis_name="c", subcore_axis_name="s")
    if transpose_out:
        out_type = jax.ShapeDtypeStruct((T, NP), jnp.float32)
    else:
        out_type = jax.ShapeDtypeStruct((NP, D), jnp.float32)
    scratch = [
        pltpu.VMEM((16,), jnp.float32),
        pltpu.VMEM((RCH, D), jnp.float32),
        pltpu.VMEM((RCH, D), jnp.float32),
        pltpu.VMEM((RCH, D), jnp.float32),
        pltpu.VMEM((RCH, D), jnp.float32),
        pltpu.VMEM((RCH, D), jnp.float32),
        pltpu.VMEM((RCH, D), jnp.float32),
        pltpu.VMEM((RCH,), jnp.float32),
        pltpu.VMEM((RCH,), jnp.float32),
        pltpu.SemaphoreType.DMA,
    ]
    return pl.kernel(functools.partial(_combine_body, transpose_out),
                     out_type=out_type, mesh=mesh, scratch_types=scratch,
                     compiler_params=pltpu.CompilerParams(
                         use_tc_tiling_on_sc=False,
                         needs_layout_passes=False))


@jax.jit
def kernel(x, edge_index, alpha1_0, alpha2_0, gamma_0, bias_0,
           alpha1_1, alpha2_1, gamma_1, bias_1):
    src2 = edge_index[0].reshape(NCHUNK, C)
    dst2 = edge_index[1].reshape(NCHUNK, C)
    xf = jnp.zeros((NP, D), jnp.float32).at[:N, :T].set(x.T)
    z = jnp.zeros((NP, D), jnp.float32)
    zdr = jnp.zeros((1, K, C, D), jnp.float32)

    pv0 = jnp.concatenate([alpha1_0[0], gamma_0[0], bias_0[0],
                           jnp.zeros((13,), jnp.float32)])
    pv1 = jnp.concatenate([alpha1_1[0], gamma_1[0], bias_1[0],
                           jnp.zeros((13,), jnp.float32)])

    s0 = _make_edge_pass(True)(xf, src2, dst2, z, zdr)
    y1 = _make_combine(False)(pv0, xf, s0, s0)
    s1 = _make_edge_pass(False)(y1, src2, dst2, z, zdr)
    y2t = _make_combine(True)(pv1, y1, s1, s0)

    return y2t[:, :N]


# speedup vs baseline: 109.7967x; 1.1223x over previous
import functools

import jax
import jax.numpy as jnp
from jax import lax
from jax.experimental import pallas as pl
from jax.experimental.pallas import tpu as pltpu
from jax.experimental.pallas import tpu_sc as plsc

N = 100000
E = 3200000
T = 4
D = 16

NC = 2
NS = 16
NW = NC * NS
C = 128
K = 4
NCHUNK = E // C
CPT = NCHUNK // NW
XTRA = NCHUNK - CPT * NW
NBF = CPT // K
CREM = CPT - NBF * K
NP = 100352
NP16 = NP // NS
RW = NP * D // 128

IBYTES = 2 * K * C * 4
GBYTES = K * C * D * 4

NTR = NP // NW
RCH = 448
NCH = NTR // RCH
LN2 = 0.6931471805599453


NLAST = N - 31 * NTR


def _pack_body(x_hbm, out_hbm, xb4, yb, sem):
    c = lax.axis_index("c")
    s = lax.axis_index("s")
    wid = c * NS + s
    base0 = wid * NTR
    iot = lax.iota(jnp.int32, 16)
    zv = jnp.zeros((16,), jnp.float32)

    def zrow(i, cr):
        yb[i, :] = zv
        return cr
    lax.fori_loop(0, RCH, zrow, 0)

    @pl.when(wid == NW - 1)
    def _():
        for t in range(T):
            for i in range(NLAST // 16, NTR // 16):
                xb4[t, pl.ds(i * 16, 16)] = zv

    for t in range(T):
        @pl.when(wid < NW - 1)
        def _(t=t):
            pltpu.sync_copy(x_hbm.at[t, pl.ds(base0, NTR)], xb4.at[t])

        @pl.when(wid == NW - 1)
        def _(t=t):
            pltpu.sync_copy(x_hbm.at[t, pl.ds(base0, NLAST)],
                            xb4.at[t, pl.ds(0, NLAST)])

    def chunk(ch, carry):
        r0 = ch * RCH
        for gi in range(RCH // 16):
            idr = gi * 16 + iot
            for t in range(T):
                xt = xb4[t, pl.ds(r0 + gi * 16, 16)]
                plsc.store_scatter(yb, [idr, jnp.full((16,), t, jnp.int32)],
                                   xt)
        pltpu.sync_copy(yb, out_hbm.at[pl.ds(base0 + r0, RCH)])
        return carry

    lax.fori_loop(0, NCH, chunk, 0)


_pack = pl.kernel(
    _pack_body,
    out_type=jax.ShapeDtypeStruct((NP, D), jnp.float32),
    mesh=plsc.VectorSubcoreMesh(core_axis_name="c", subcore_axis_name="s"),
    scratch_types=[
        pltpu.VMEM((T, NTR), jnp.float32),
        pltpu.VMEM((RCH, D), jnp.float32),
        pltpu.SemaphoreType.DMA,
    ],
    compiler_params=pltpu.CompilerParams(use_tc_tiling_on_sc=False,
                                         needs_layout_passes=False),
)


def _edge_pass_body(with_deg, *refs):
    if with_deg:
        (x_hbm, src_hbm, dst_hbm, z_hbm, zdr_hbm,
         s_out,
         acc_sp, src_blk, dst_blk, rows, tail_s, tail_d, cones,
         isem, gsem, ssem) = refs
    else:
        (x_hbm, src_hbm, dst_hbm, z_hbm, zdr_hbm,
         s_out,
         acc_sp, src_blk, dst_blk, rows, tail_s, tail_d,
         isem, gsem, ssem) = refs
        cones = None

    def drain_scatters(_):
        pltpu.make_async_copy(zdr_hbm.at[0], rows.at[0], ssem).wait()
        if with_deg:
            pltpu.make_async_copy(zdr_hbm.at[0], rows.at[0], ssem).wait()

    c = lax.axis_index("c")
    s = lax.axis_index("s")
    wid = c * NS + s
    row0 = s * NP16
    cb0 = wid * CPT

    pltpu.sync_copy(z_hbm.at[pl.ds(row0, NP16)], acc_sp.at[pl.ds(row0, NP16)])
    if with_deg:
        cvec = jnp.where(lax.iota(jnp.int32, 16) == 4,
                         jnp.float32(1.0), jnp.float32(0.0))
        for i in range(C):
            cones[i, :] = cvec
    plsc.subcore_barrier()

    pltpu.async_copy(src_hbm.at[pl.ds(cb0, K)], src_blk.at[0], isem)
    pltpu.async_copy(dst_hbm.at[pl.ds(cb0, K)], dst_blk.at[0], isem)

    def block(nb, carry):
        b2 = lax.rem(nb, 2)
        b3 = lax.rem(nb, 3)
        pltpu.make_async_copy(src_hbm.at[pl.ds(cb0, K)],
                              src_blk.at[b3], isem).wait()
        pltpu.make_async_copy(src_hbm.at[pl.ds(cb0, K)],
                              dst_blk.at[b3], isem).wait()

        pl.when(nb >= 2)(lambda: drain_scatters(0))

        for j in range(K):
            pltpu.async_copy(x_hbm.at[src_blk.at[b3, j]],
                             rows.at[b2, j], gsem)

        @pl.when(nb + 1 < NBF)
        def _():
            nxt = cb0 + (nb + 1) * K
            b3n = lax.rem(nb + 1, 3)
            pltpu.async_copy(src_hbm.at[pl.ds(nxt, K)], src_blk.at[b3n], isem)
            pltpu.async_copy(dst_hbm.at[pl.ds(nxt, K)], dst_blk.at[b3n], isem)

        pltpu.make_async_copy(zdr_hbm.at[0], rows.at[0], gsem).wait()
        for j in range(K):
            pltpu.async_copy(rows.at[b2, j],
                             acc_sp.at[dst_blk.at[b3, j]], ssem, add=True)
        if with_deg:
            for j in range(K):
                pltpu.async_copy(cones,
                                 acc_sp.at[src_blk.at[b3, j]], ssem, add=True)
        return carry

    lax.fori_loop(0, NBF, block, 0)
    drain_scatters(0)
    drain_scatters(0)

    for j in range(K):
        for i in range(C // 16):
            dummy = N + jnp.bitwise_and(i * 16 + lax.iota(jnp.int32, 16), 255)
            tail_s[j, pl.ds(i * 16, 16)] = dummy
            tail_d[j, pl.ds(i * 16, 16)] = dummy
    pltpu.sync_copy(src_hbm.at[pl.ds(cb0 + NBF * K, CREM)],
                    tail_s.at[pl.ds(0, CREM)])
    pltpu.sync_copy(dst_hbm.at[pl.ds(cb0 + NBF * K, CREM)],
                    tail_d.at[pl.ds(0, CREM)])

    @pl.when(wid < XTRA)
    def _():
        xc = NW * CPT + wid
        pltpu.sync_copy(src_hbm.at[pl.ds(xc, 1)], tail_s.at[pl.ds(CREM, 1)])
        pltpu.sync_copy(dst_hbm.at[pl.ds(xc, 1)], tail_d.at[pl.ds(CREM, 1)])

    gds = []
    for j in range(K):
        gds.append(pltpu.async_copy(x_hbm.at[tail_s.at[j]],
                                    rows.at[0, j], gsem))
    for d in gds:
        d.wait()
    sds = []
    for j in range(K):
        sds.append(pltpu.async_copy(rows.at[0, j],
                                    acc_sp.at[tail_d.at[j]], ssem, add=True))
    if with_deg:
        for j in range(K):
            sds.append(pltpu.async_copy(cones,
                                        acc_sp.at[tail_s.at[j]], ssem,
                                        add=True))
    for d in sds:
        d.wait()

    plsc.subcore_barrier()

    pltpu.sync_copy(acc_sp.at[pl.ds(row0, NP16)], s_out.at[c, pl.ds(row0, NP16)])


def _make_edge_pass(with_deg):
    mesh = plsc.VectorSubcoreMesh(core_axis_name="c", subcore_axis_name="s")
    scratch = [
        pltpu.VMEM_SHARED((NP, D), jnp.float32),
        pltpu.VMEM((3, K, C), jnp.int32),
        pltpu.VMEM((3, K, C), jnp.int32),
        pltpu.VMEM((2, K, C, D), jnp.float32),
        pltpu.VMEM((K, C), jnp.int32),
        pltpu.VMEM((K, C), jnp.int32),
    ]
    if with_deg:
        scratch.append(pltpu.VMEM((C, D), jnp.float32))
    scratch += [pltpu.SemaphoreType.DMA,
                pltpu.SemaphoreType.DMA,
                pltpu.SemaphoreType.DMA]
    return pl.kernel(functools.partial(_edge_pass_body, with_deg),
                     out_type=jax.ShapeDtypeStruct((NC, NP, D), jnp.float32),
                     mesh=mesh, scratch_types=scratch,
                     compiler_params=pltpu.CompilerParams(
                         use_tc_tiling_on_sc=False))


def _ln(d):
    bits = plsc.bitcast(d, jnp.int32)
    e = (bits >> 23) - 127
    m = plsc.bitcast(jnp.bitwise_or(jnp.bitwise_and(bits, 0x007FFFFF),
                                    0x3F800000), jnp.float32)
    r = (m - 1.0) / (m + 1.0)
    r2 = r * r
    at = r * (1.0 + r2 * (1.0 / 3.0 + r2 * (1.0 / 5.0 + r2 * (1.0 / 7.0
                                                              + r2 / 9.0))))
    return e.astype(jnp.float32) * LN2 + 2.0 * at


def _combine_body(transpose_out, pv_hbm, x_hbm, sm_hbm, sd_hbm, out_hbm,
                  pbuf, xb, m0, m1, d0, d1, yb, wsb, wnb, sem):
    c = lax.axis_index("c")
    s = lax.axis_index("s")
    wid = c * NS + s
    base0 = wid * NTR

    pltpu.sync_copy(pv_hbm, pbuf)
    pvv = pbuf[...]
    a1v = jnp.full((16,), pvv[0], jnp.float32)
    gv = jnp.full((16,), pvv[1], jnp.float32)
    bv = jnp.full((16,), pvv[2], jnp.float32)
    pv = 1.0 / (1.0 + jnp.exp(-gv))
    swv = jnp.exp(a1v)
    e2a = jnp.exp(2.0 * a1v)
    nwv = swv * (e2a - 1.0) / (e2a + 1.0)

    idc4 = jnp.full((16,), 4, jnp.int32)
    iot = lax.iota(jnp.int32, 16)

    def chunk(ch, carry):
        base = base0 + ch * RCH
        cps = [pltpu.async_copy(x_hbm.at[pl.ds(base, RCH)], xb, sem),
               pltpu.async_copy(sm_hbm.at[0, pl.ds(base, RCH)], m0, sem),
               pltpu.async_copy(sm_hbm.at[1, pl.ds(base, RCH)], m1, sem),
               pltpu.async_copy(sd_hbm.at[0, pl.ds(base, RCH)], d0, sem),
               pltpu.async_copy(sd_hbm.at[1, pl.ds(base, RCH)], d1, sem)]
        for cp in cps:
            cp.wait()
        for gi in range(RCH // 16):
            idr = gi * 16 + iot
            dv = (plsc.load_gather(d0, [idr, idc4])
                  + plsc.load_gather(d1, [idr, idc4]))
            lnd = _ln(jnp.maximum(dv, 1.0))
            wsb[pl.ds(gi * 16, 16)] = swv * jnp.exp(pv * lnd)
            wnb[pl.ds(gi * 16, 16)] = nwv * jnp.exp((pv - 1.0) * lnd)

        def rowgrp(gi2, cr):
            r0 = gi2 * 16
            wsv = wsb[pl.ds(r0, 16)]
            wnv = wnb[pl.ds(r0, 16)]
            for l in range(16):
                i = r0 + l
                srow = m0[i, :] + m1[i, :]
                ws = jnp.full((16,), wsv[l], jnp.float32)
                wn = jnp.full((16,), wnv[l], jnp.float32)
                yb[i, :] = ws * xb[i, :] + wn * srow + bv
            return cr

        lax.fori_loop(0, RCH // 16, rowgrp, 0)

        if not transpose_out:
            pltpu.sync_copy(yb, out_hbm.at[pl.ds(base, RCH)])
        else:
            for t in range(T):
                idct = jnp.full((16,), t, jnp.int32)
                for gi in range(RCH // 16):
                    idr = gi * 16 + iot
                    wsb[pl.ds(gi * 16, 16)] = plsc.load_gather(yb, [idr, idct])
                pltpu.sync_copy(wsb, out_hbm.at[t, pl.ds(base, RCH)])
        return carry

    lax.fori_loop(0, NCH, chunk, 0)


def _make_combine(transpose_out):
    mesh = plsc.VectorSubcoreMesh(core_axis_name="c", subcore_axis_name="s")
    if transpose_out:
        out_type = jax.ShapeDtypeStruct((T, NP), jnp.float32)
    else:
        out_type = jax.ShapeDtypeStruct((NP, D), jnp.float32)
    scratch = [
        pltpu.VMEM((16,), jnp.float32),
        pltpu.VMEM((RCH, D), jnp.float32),
        pltpu.VMEM((RCH, D), jnp.float32),
        pltpu.VMEM((RCH, D), jnp.float32),
        pltpu.VMEM((RCH, D), jnp.float32),
        pltpu.VMEM((RCH, D), jnp.float32),
        pltpu.VMEM((RCH, D), jnp.float32),
        pltpu.VMEM((RCH,), jnp.float32),
        pltpu.VMEM((RCH,), jnp.float32),
        pltpu.SemaphoreType.DMA,
    ]
    return pl.kernel(functools.partial(_combine_body, transpose_out),
                     out_type=out_type, mesh=mesh, scratch_types=scratch,
                     compiler_params=pltpu.CompilerParams(
                         use_tc_tiling_on_sc=False,
                         needs_layout_passes=False))


@jax.jit
def kernel(x, edge_index, alpha1_0, alpha2_0, gamma_0, bias_0,
           alpha1_1, alpha2_1, gamma_1, bias_1):
    src2 = edge_index[0].reshape(NCHUNK, C)
    dst2 = edge_index[1].reshape(NCHUNK, C)
    xf = _pack(x)
    z = jnp.zeros((NP, D), jnp.float32)
    zdr = jnp.zeros((1, K, C, D), jnp.float32)

    pv0 = jnp.concatenate([alpha1_0[0], gamma_0[0], bias_0[0],
                           jnp.zeros((13,), jnp.float32)])
    pv1 = jnp.concatenate([alpha1_1[0], gamma_1[0], bias_1[0],
                           jnp.zeros((13,), jnp.float32)])

    s0 = _make_edge_pass(True)(xf, src2, dst2, z, zdr)
    y1 = _make_combine(False)(pv0, xf, s0, s0)
    s1 = _make_edge_pass(False)(y1, src2, dst2, z, zdr)
    y2t = _make_combine(True)(pv1, y1, s1, s0)

    return y2t[:, :N]
